# Initial kernel scaffold; baseline (speedup 1.0000x reference)
import jax
import jax.numpy as jnp
from jax.experimental import pallas as pl


def kernel(x, table):
    raise NotImplementedError("write your pallas kernel here")


# speedup vs baseline: 1.7953x; 1.7953x over previous
import functools

import jax
import jax.numpy as jnp
from jax import lax
from jax.experimental import pallas as pl
from jax.experimental.pallas import tpu as pltpu
from jax.experimental.pallas import tpu_sc as plsc

D_MODEL = 64
NUM_CORES = 2
NUM_SUBCORES = 16
NUM_WORKERS = NUM_CORES * NUM_SUBCORES
CHUNK = 512


@functools.partial(jax.jit, static_argnums=(2,))
def _gather_rows(idx, table, n_rows):
    n_per_w = n_rows // NUM_WORKERS
    n_chunks = n_per_w // CHUNK
    mesh = plsc.VectorSubcoreMesh(core_axis_name="c", subcore_axis_name="s")

    @functools.partial(
        pl.kernel,
        mesh=mesh,
        out_type=jax.ShapeDtypeStruct((n_rows, D_MODEL), jnp.float32),
        scratch_types=[
            pltpu.VMEM((CHUNK,), jnp.int32),
            pltpu.VMEM((CHUNK, D_MODEL), jnp.float32),
            pltpu.SemaphoreType.DMA,
        ],
        compiler_params=pltpu.CompilerParams(use_tc_tiling_on_sc=False),
    )
    def k(idx_hbm, table_hbm, out_hbm, idx_v, rows_v, sem):
        wid = lax.axis_index("s") * NUM_CORES + lax.axis_index("c")
        base = wid * n_per_w

        def body(i, carry):
            off = base + i * CHUNK
            pltpu.sync_copy(idx_hbm.at[pl.ds(off, CHUNK)], idx_v)
            pltpu.async_copy(table_hbm.at[idx_v], rows_v, sem).wait()
            pltpu.sync_copy(rows_v, out_hbm.at[pl.ds(off, CHUNK)])
            return carry

        lax.fori_loop(0, n_chunks, body, 0)

    return k(idx, table)


def kernel(x, table):
    b, s = x.shape
    n_rows = b * s
    idx = x.reshape(n_rows).astype(jnp.int32)
    out = _gather_rows(idx, table, n_rows)
    return out.reshape(b, s, D_MODEL)


# speedup vs baseline: 1.8756x; 1.0447x over previous
import functools

import jax
import jax.numpy as jnp
from jax import lax
from jax.experimental import pallas as pl
from jax.experimental.pallas import tpu as pltpu
from jax.experimental.pallas import tpu_sc as plsc

D_MODEL = 64
NUM_CORES = 2
NUM_SUBCORES = 16
NUM_WORKERS = NUM_CORES * NUM_SUBCORES
CHUNK = 512


@functools.partial(jax.jit, static_argnums=(2,))
def _gather_rows(idx, table, n_rows):
    n_per_w = n_rows // NUM_WORKERS
    n_chunks = n_per_w // CHUNK
    assert n_chunks % 2 == 0
    idx3 = idx.reshape(NUM_WORKERS, n_chunks, CHUNK)
    mesh = plsc.VectorSubcoreMesh(core_axis_name="c", subcore_axis_name="s")

    @functools.partial(
        pl.kernel,
        mesh=mesh,
        out_type=jax.ShapeDtypeStruct((n_rows, D_MODEL), jnp.float32),
        scratch_types=[
            pltpu.VMEM((n_chunks, CHUNK), jnp.int32),
            pltpu.VMEM((2, CHUNK, D_MODEL), jnp.float32),
            pltpu.SemaphoreType.DMA,
            pltpu.SemaphoreType.DMA,
        ],
        compiler_params=pltpu.CompilerParams(use_tc_tiling_on_sc=False),
    )
    def k(idx_hbm, table_hbm, out_hbm, idx_v, rows_v, g_sem, o_sem):
        wid = lax.axis_index("s") * NUM_CORES + lax.axis_index("c")
        base = wid * n_per_w
        pltpu.sync_copy(idx_hbm.at[wid], idx_v)
        pltpu.async_copy(table_hbm.at[idx_v.at[0]], rows_v.at[0], g_sem)

        def step(i, s, s_next):
            @pl.when(i >= 1)
            def _():
                pltpu.make_async_copy(
                    rows_v.at[s_next],
                    out_hbm.at[pl.ds(base, CHUNK)],
                    o_sem,
                ).wait()

            @pl.when(i + 1 < n_chunks)
            def _():
                pltpu.async_copy(
                    table_hbm.at[idx_v.at[i + 1]], rows_v.at[s_next], g_sem
                )

            pltpu.make_async_copy(
                table_hbm.at[idx_v.at[i]], rows_v.at[s], g_sem
            ).wait()
            pltpu.async_copy(
                rows_v.at[s], out_hbm.at[pl.ds(base + i * CHUNK, CHUNK)], o_sem
            )

        def body(p, carry):
            step(2 * p, 0, 1)
            step(2 * p + 1, 1, 0)
            return carry

        lax.fori_loop(0, n_chunks // 2, body, 0)
        pltpu.make_async_copy(
            rows_v.at[1], out_hbm.at[pl.ds(base, CHUNK)], o_sem
        ).wait()

    return k(idx3, table)


def kernel(x, table):
    b, s = x.shape
    n_rows = b * s
    idx = x.reshape(n_rows).astype(jnp.int32)
    out = _gather_rows(idx, table, n_rows)
    return out.reshape(b, s, D_MODEL)
